# Initial kernel scaffold; baseline (speedup 1.0000x reference)
import jax
import jax.numpy as jnp
from jax.experimental import pallas as pl


def kernel(x, edge_index, batch, W_in, b_in, W1, b1, W2, b2, Wfc1, bfc1, Wfc2, bfc2):
    raise NotImplementedError("write your pallas kernel here")


# speedup vs baseline: 5.3691x; 5.3691x over previous
import functools

import jax
import jax.numpy as jnp
from jax import lax
from jax.experimental import pallas as pl
from jax.experimental.pallas import tpu as pltpu
from jax.experimental.pallas import tpu_sc as plsc

N_NODES = 10000
N_EDGES = 320000
E = 2 * N_EDGES
D_IN = 128
D_HID = 64
D_OUT = 32
N_CLASSES = 2
N_GRAPHS = 64

CHUNK = 128
NUM_CHUNKS = E // CHUNK
N_PAD = 10240


def _make_agg():
    info = plsc.get_sparse_core_info()
    NC, NS, L = info.num_cores, info.num_subcores, info.num_lanes
    NW = NC * NS
    ROWS_PER_TILE = N_PAD // NS
    ZROWS = 128
    chunks_per_worker = (NUM_CHUNKS + NW - 1) // NW

    mesh = plsc.VectorSubcoreMesh(core_axis_name="c", subcore_axis_name="s")

    @functools.partial(
        pl.kernel,
        mesh=mesh,
        compiler_params=pltpu.CompilerParams(use_tc_tiling_on_sc=False),
        out_type=jax.ShapeDtypeStruct((NC, N_PAD, D_HID), jnp.float32),
        scratch_types=[
            pltpu.VMEM((CHUNK,), jnp.int32),
            pltpu.VMEM((CHUNK,), jnp.int32),
            pltpu.VMEM((CHUNK,), jnp.int32),
            pltpu.VMEM((CHUNK, D_HID), jnp.float32),
            pltpu.VMEM((ZROWS, D_HID), jnp.float32),
            pltpu.VMEM_SHARED((N_PAD, D_HID), jnp.float32),
            pltpu.SemaphoreType.DMA,
        ],
    )
    def agg(g_hbm, src_hbm, dst_hbm, wfl_hbm, out_hbm,
            src_v, dst_v, wfl_v, rows_v, zero_v, acc_sh, sem):
        cid = lax.axis_index("c")
        sid = lax.axis_index("s")
        wid = sid * NC + cid

        zeros16 = jnp.zeros((L,), jnp.float32)

        def zbody(i, _):
            r = i // (D_HID // L)
            c = (i % (D_HID // L)) * L
            zero_v[r, pl.ds(c, L)] = zeros16
            return 0

        lax.fori_loop(0, ZROWS * (D_HID // L), zbody, 0)
        row0 = sid * ROWS_PER_TILE
        for j in range(ROWS_PER_TILE // ZROWS):
            pltpu.sync_copy(zero_v, acc_sh.at[pl.ds(row0 + j * ZROWS, ZROWS)])
        plsc.subcore_barrier()

        def chunk_body(i, _):
            chunk_id = wid + i * NW

            @pl.when(chunk_id < NUM_CHUNKS)
            def _():
                base = chunk_id * CHUNK
                pltpu.sync_copy(src_hbm.at[pl.ds(base, CHUNK)], src_v)
                pltpu.sync_copy(dst_hbm.at[pl.ds(base, CHUNK)], dst_v)
                pltpu.sync_copy(wfl_hbm.at[pl.ds(base, CHUNK)], wfl_v)
                for g in range(CHUNK // L):
                    sl = pl.ds(g * L, L)
                    src_v[sl] = jnp.where(wfl_v[sl] != 0, src_v[sl], N_NODES)
                pltpu.async_copy(g_hbm.at[src_v], rows_v, sem).wait()
                pltpu.sync_copy(rows_v, acc_sh.at[dst_v], add=True)

            return 0

        lax.fori_loop(0, chunks_per_worker, chunk_body, 0)
        plsc.subcore_barrier()

        for j in range(ROWS_PER_TILE // ZROWS):
            r = row0 + j * ZROWS
            pltpu.sync_copy(acc_sh.at[pl.ds(r, ZROWS)],
                            out_hbm.at[cid, pl.ds(r, ZROWS)])

    return agg, NC


_AGG, _NC = _make_agg()


def _dense1_body(x_ref, win_ref, bin_ref, w1_ref, deg_ref, g1_ref, dinv_ref):
    h0 = jnp.maximum(
        jnp.dot(x_ref[...], win_ref[...], preferred_element_type=jnp.float32)
        + bin_ref[...], 0.0)
    hw1 = jnp.dot(h0, w1_ref[...], preferred_element_type=jnp.float32)
    dinv = lax.rsqrt(jnp.maximum(deg_ref[...], 1.0))
    g1_ref[...] = dinv * hw1
    dinv_ref[...] = dinv


def _dense2_body(sp_ref, g1_ref, dinv_ref, b1_ref, w2_ref, g2_ref):
    agg = sp_ref[0, :N_NODES] + sp_ref[1, :N_NODES] + g1_ref[...]
    h2 = jnp.maximum(dinv_ref[...] * agg + b1_ref[...], 0.0)
    hw2 = jnp.dot(h2, w2_ref[...], preferred_element_type=jnp.float32)
    g2_ref[...] = dinv_ref[...] * hw2


def _dense3_body(sp_ref, g2_ref, dinv_ref, b2_ref, bat_ref,
                 wf1_ref, bf1_ref, wf2_ref, bf2_ref, out_ref):
    agg = sp_ref[0, :N_NODES] + sp_ref[1, :N_NODES] + g2_ref[...]
    h3 = jnp.maximum(dinv_ref[...] * agg + b2_ref[...], 0.0)
    gid = lax.broadcasted_iota(jnp.int32, (1, N_GRAPHS), 1)
    P = (bat_ref[...] == gid).astype(jnp.float32)
    sums = lax.dot_general(P, h3, (((0,), (0,)), ((), ())),
                           preferred_element_type=jnp.float32)
    ones = jnp.ones((N_NODES, 1), jnp.float32)
    cnts = lax.dot_general(P, ones, (((0,), (0,)), ((), ())),
                           preferred_element_type=jnp.float32)
    mean = sums / jnp.maximum(cnts, 1.0)
    z1 = jnp.maximum(
        jnp.dot(mean, wf1_ref[...], preferred_element_type=jnp.float32)
        + bf1_ref[...], 0.0)
    z2 = (jnp.dot(z1, wf2_ref[...], preferred_element_type=jnp.float32)
          + bf2_ref[...])
    m = jnp.max(z2, axis=1, keepdims=True)
    lse = jnp.log(jnp.sum(jnp.exp(z2 - m), axis=1, keepdims=True)) + m
    out_ref[...] = z2 - lse


def _dense1(x, W_in, b_in, W1, deg):
    return pl.pallas_call(
        _dense1_body,
        out_shape=(jax.ShapeDtypeStruct((N_NODES, D_HID), jnp.float32),
                   jax.ShapeDtypeStruct((N_NODES, 1), jnp.float32)),
    )(x, W_in, b_in.reshape(1, D_HID), W1, deg)


def _dense2(Sp, g1, dinv, b1, W2):
    return pl.pallas_call(
        _dense2_body,
        out_shape=jax.ShapeDtypeStruct((N_NODES, D_HID), jnp.float32),
    )(Sp, g1, dinv, b1.reshape(1, D_HID), W2)


def _dense3(Sp, g2, dinv, b2, batch_col, Wfc1, bfc1, Wfc2, bfc2):
    return pl.pallas_call(
        _dense3_body,
        out_shape=jax.ShapeDtypeStruct((N_GRAPHS, N_CLASSES), jnp.float32),
    )(Sp, g2, dinv, b2.reshape(1, D_HID), batch_col,
      Wfc1, bfc1.reshape(1, D_OUT), Wfc2, bfc2.reshape(1, N_CLASSES))


def kernel(x, edge_index, batch, W_in, b_in, W1, b1, W2, b2,
           Wfc1, bfc1, Wfc2, bfc2):
    src0 = edge_index[0].astype(jnp.int32)
    dst0 = edge_index[1].astype(jnp.int32)
    s = jnp.concatenate([src0, dst0])
    d = jnp.concatenate([dst0, src0])
    sk = jnp.sort(s * N_NODES + d)
    first = jnp.concatenate([jnp.ones((1,), bool), sk[1:] != sk[:-1]])
    srt_src = sk // N_NODES
    srt_dst = sk % N_NODES
    wfl = first.astype(jnp.int32)

    cw0 = jnp.concatenate([jnp.zeros((1,), jnp.float32),
                           jnp.cumsum(first.astype(jnp.float32))])
    bounds = jnp.searchsorted(srt_src, jnp.arange(N_NODES + 1, dtype=jnp.int32))
    deg = (cw0[bounds[1:]] - cw0[bounds[:-1]] + 1.0).reshape(N_NODES, 1)

    g1, dinv = _dense1(x, W_in, b_in, W1, deg)
    gp1 = jnp.concatenate([g1, jnp.zeros((1, D_HID), jnp.float32)])
    Sp1 = _AGG(gp1, srt_src, srt_dst, wfl)

    g2 = _dense2(Sp1, g1, dinv, b1, W2)
    gp2 = jnp.concatenate([g2, jnp.zeros((1, D_HID), jnp.float32)])
    Sp2 = _AGG(gp2, srt_src, srt_dst, wfl)

    batch_col = batch.astype(jnp.int32).reshape(N_NODES, 1)
    return _dense3(Sp2, g2, dinv, b2, batch_col, Wfc1, bfc1, Wfc2, bfc2)


# speedup vs baseline: 7.1889x; 1.3389x over previous
import functools

import jax
import jax.numpy as jnp
from jax import lax
from jax.experimental import pallas as pl
from jax.experimental.pallas import tpu as pltpu
from jax.experimental.pallas import tpu_sc as plsc

N_NODES = 10000
N_EDGES = 320000
E = 2 * N_EDGES
D_IN = 128
D_HID = 64
D_OUT = 32
N_CLASSES = 2
N_GRAPHS = 64

CHUNK = 128
N_PAD = 10240
NCPW = 158
NUM_CHUNKS_PAD = 32 * NCPW
E_PAD = NUM_CHUNKS_PAD * CHUNK


def _make_agg():
    info = plsc.get_sparse_core_info()
    NC, NS, L = info.num_cores, info.num_subcores, info.num_lanes
    NW = NC * NS
    ROWS_PER_TILE = N_PAD // NS
    ZROWS = 128

    mesh = plsc.VectorSubcoreMesh(core_axis_name="c", subcore_axis_name="s")

    @functools.partial(
        pl.kernel,
        mesh=mesh,
        compiler_params=pltpu.CompilerParams(use_tc_tiling_on_sc=False),
        out_type=jax.ShapeDtypeStruct((NC, N_PAD, D_HID), jnp.float32),
        scratch_types=[
            pltpu.VMEM((NCPW, CHUNK), jnp.int32),
            pltpu.VMEM((NCPW, CHUNK), jnp.int32),
            pltpu.VMEM((CHUNK, D_HID), jnp.float32),
            pltpu.VMEM((CHUNK, D_HID), jnp.float32),
            pltpu.VMEM((ZROWS, D_HID), jnp.float32),
            pltpu.VMEM_SHARED((N_PAD, D_HID), jnp.float32),
            pltpu.SemaphoreType.DMA,
            pltpu.SemaphoreType.DMA,
        ],
    )
    def agg(g_hbm, src_hbm, dst_hbm, out_hbm,
            src_v, dst_v, rows0, rows1, zero_v, acc_sh, sem0, sem1):
        cid = lax.axis_index("c")
        sid = lax.axis_index("s")
        wid = sid * NC + cid

        pltpu.sync_copy(src_hbm.at[pl.ds(wid * NCPW, NCPW)], src_v)
        pltpu.sync_copy(dst_hbm.at[pl.ds(wid * NCPW, NCPW)], dst_v)

        zeros16 = jnp.zeros((L,), jnp.float32)

        def zbody(i, _):
            r = i // (D_HID // L)
            c = (i % (D_HID // L)) * L
            zero_v[r, pl.ds(c, L)] = zeros16
            return 0

        lax.fori_loop(0, ZROWS * (D_HID // L), zbody, 0)
        row0 = sid * ROWS_PER_TILE
        for j in range(ROWS_PER_TILE // ZROWS):
            pltpu.sync_copy(zero_v, acc_sh.at[pl.ds(row0 + j * ZROWS, ZROWS)])
        plsc.subcore_barrier()

        pltpu.async_copy(g_hbm.at[src_v.at[0]], rows0, sem0)

        def chunk_body(j, _):
            i0 = 2 * j
            pltpu.async_copy(g_hbm.at[src_v.at[i0 + 1]], rows1, sem1)
            pltpu.make_async_copy(g_hbm.at[src_v.at[i0]], rows0, sem0).wait()
            pltpu.sync_copy(rows0, acc_sh.at[dst_v.at[i0]], add=True)

            @pl.when(i0 + 2 < NCPW)
            def _():
                pltpu.async_copy(g_hbm.at[src_v.at[i0 + 2]], rows0, sem0)

            pltpu.make_async_copy(g_hbm.at[src_v.at[i0 + 1]], rows1, sem1).wait()
            pltpu.sync_copy(rows1, acc_sh.at[dst_v.at[i0 + 1]], add=True)
            return 0

        lax.fori_loop(0, NCPW // 2, chunk_body, 0)
        plsc.subcore_barrier()

        for j in range(ROWS_PER_TILE // ZROWS):
            r = row0 + j * ZROWS
            pltpu.sync_copy(acc_sh.at[pl.ds(r, ZROWS)],
                            out_hbm.at[cid, pl.ds(r, ZROWS)])

    return agg, NC


_AGG, _NC = _make_agg()


def _dense1_body(x_ref, win_ref, bin_ref, w1_ref, deg_ref, g1_ref, dinv_ref):
    h0 = jnp.maximum(
        jnp.dot(x_ref[...], win_ref[...], preferred_element_type=jnp.float32)
        + bin_ref[...], 0.0)
    hw1 = jnp.dot(h0, w1_ref[...], preferred_element_type=jnp.float32)
    dinv = lax.rsqrt(jnp.maximum(deg_ref[...], 1.0))
    g1_ref[...] = dinv * hw1
    dinv_ref[...] = dinv


def _dense2_body(sp_ref, g1_ref, dinv_ref, b1_ref, w2_ref, g2_ref):
    agg = sp_ref[0, :N_NODES] + sp_ref[1, :N_NODES] + g1_ref[...]
    h2 = jnp.maximum(dinv_ref[...] * agg + b1_ref[...], 0.0)
    hw2 = jnp.dot(h2, w2_ref[...], preferred_element_type=jnp.float32)
    g2_ref[...] = dinv_ref[...] * hw2


def _dense3_body(sp_ref, g2_ref, dinv_ref, b2_ref, bat_ref,
                 wf1_ref, bf1_ref, wf2_ref, bf2_ref, out_ref):
    agg = sp_ref[0, :N_NODES] + sp_ref[1, :N_NODES] + g2_ref[...]
    h3 = jnp.maximum(dinv_ref[...] * agg + b2_ref[...], 0.0)
    gid = lax.broadcasted_iota(jnp.int32, (1, N_GRAPHS), 1)
    P = (bat_ref[...] == gid).astype(jnp.float32)
    sums = lax.dot_general(P, h3, (((0,), (0,)), ((), ())),
                           preferred_element_type=jnp.float32)
    ones = jnp.ones((N_NODES, 1), jnp.float32)
    cnts = lax.dot_general(P, ones, (((0,), (0,)), ((), ())),
                           preferred_element_type=jnp.float32)
    mean = sums / jnp.maximum(cnts, 1.0)
    z1 = jnp.maximum(
        jnp.dot(mean, wf1_ref[...], preferred_element_type=jnp.float32)
        + bf1_ref[...], 0.0)
    z2 = (jnp.dot(z1, wf2_ref[...], preferred_element_type=jnp.float32)
          + bf2_ref[...])
    m = jnp.max(z2, axis=1, keepdims=True)
    lse = jnp.log(jnp.sum(jnp.exp(z2 - m), axis=1, keepdims=True)) + m
    out_ref[...] = z2 - lse


def _dense1(x, W_in, b_in, W1, deg):
    return pl.pallas_call(
        _dense1_body,
        out_shape=(jax.ShapeDtypeStruct((N_NODES, D_HID), jnp.float32),
                   jax.ShapeDtypeStruct((N_NODES, 1), jnp.float32)),
    )(x, W_in, b_in.reshape(1, D_HID), W1, deg)


def _dense2(Sp, g1, dinv, b1, W2):
    return pl.pallas_call(
        _dense2_body,
        out_shape=jax.ShapeDtypeStruct((N_NODES, D_HID), jnp.float32),
    )(Sp, g1, dinv, b1.reshape(1, D_HID), W2)


def _dense3(Sp, g2, dinv, b2, batch_col, Wfc1, bfc1, Wfc2, bfc2):
    return pl.pallas_call(
        _dense3_body,
        out_shape=jax.ShapeDtypeStruct((N_GRAPHS, N_CLASSES), jnp.float32),
    )(Sp, g2, dinv, b2.reshape(1, D_HID), batch_col,
      Wfc1, bfc1.reshape(1, D_OUT), Wfc2, bfc2.reshape(1, N_CLASSES))


def kernel(x, edge_index, batch, W_in, b_in, W1, b1, W2, b2,
           Wfc1, bfc1, Wfc2, bfc2):
    src0 = edge_index[0].astype(jnp.int32)
    dst0 = edge_index[1].astype(jnp.int32)
    s = jnp.concatenate([src0, dst0])
    d = jnp.concatenate([dst0, src0])
    sk = jnp.sort(s * N_NODES + d)
    first = jnp.concatenate([jnp.ones((1,), bool), sk[1:] != sk[:-1]])
    srt_src = sk // N_NODES
    srt_dst = sk % N_NODES
    srcm = jnp.where(first, srt_src, N_NODES)
    srcm = jnp.concatenate(
        [srcm, jnp.full((E_PAD - E,), N_NODES, jnp.int32)]).reshape(
            NUM_CHUNKS_PAD, CHUNK)
    dstm = jnp.concatenate(
        [srt_dst, jnp.zeros((E_PAD - E,), jnp.int32)]).reshape(
            NUM_CHUNKS_PAD, CHUNK)

    cw0 = jnp.concatenate([jnp.zeros((1,), jnp.float32),
                           jnp.cumsum(first.astype(jnp.float32))])
    bounds = jnp.searchsorted(srt_src, jnp.arange(N_NODES + 1, dtype=jnp.int32))
    deg = (cw0[bounds[1:]] - cw0[bounds[:-1]] + 1.0).reshape(N_NODES, 1)

    g1, dinv = _dense1(x, W_in, b_in, W1, deg)
    gp1 = jnp.concatenate([g1, jnp.zeros((1, D_HID), jnp.float32)])
    Sp1 = _AGG(gp1, srcm, dstm)

    g2 = _dense2(Sp1, g1, dinv, b1, W2)
    gp2 = jnp.concatenate([g2, jnp.zeros((1, D_HID), jnp.float32)])
    Sp2 = _AGG(gp2, srcm, dstm)

    batch_col = batch.astype(jnp.int32).reshape(N_NODES, 1)
    return _dense3(Sp2, g2, dinv, b2, batch_col, Wfc1, bfc1, Wfc2, bfc2)
